# baseline (speedup 1.0000x reference)
import functools
import math

import jax
import jax.numpy as jnp
from jax import lax
from jax.experimental import pallas as pl
from jax.experimental.pallas import tpu as pltpu
from jax.experimental.pallas import tpu_sc as plsc

B = 16384
D = 128
VT = 1000
VTP = 1024
LT = 50

_SC_PARAMS = dict(use_tc_tiling_on_sc=True, needs_layout_passes=False)


@functools.lru_cache(maxsize=None)
def _make_sc_feed(VF, n, Bb):
  info = plsc.get_sparse_core_info()
  NC, NS = info.num_cores, info.num_subcores
  NW = NC * NS
  b_per_w = Bb // NW
  n_chunks = b_per_w // n
  mesh = plsc.VectorSubcoreMesh(core_axis_name="c", subcore_axis_name="s")

  @functools.partial(
      pl.kernel,
      out_type=(
          jax.ShapeDtypeStruct((Bb, D), jnp.float32),
          jax.ShapeDtypeStruct((Bb, D), jnp.float32),
      ),
      mesh=mesh,
      compiler_params=pltpu.CompilerParams(**_SC_PARAMS),
      scratch_types=[
          pltpu.VMEM((n,), jnp.int32),
          pltpu.VMEM((n, D), jnp.float32),
          pltpu.VMEM((n, D), jnp.float32),
          pltpu.SemaphoreType.DMA,
          pltpu.SemaphoreType.DMA,
      ],
  )
  def sc_feed(feed_id, feed_tab, w2v_tab, e1_o, e2_o, idx_v, r1, r2, s1, s2):
    wid = lax.axis_index("s") * NC + lax.axis_index("c")
    base = wid * b_per_w

    def body(j, carry):
      off = base + j * n
      pltpu.sync_copy(feed_id.at[pl.ds(off, n)], idx_v)
      c1 = pltpu.async_copy(feed_tab.at[idx_v], r1, s1)
      c2 = pltpu.async_copy(w2v_tab.at[idx_v], r2, s2)
      c1.wait()
      pltpu.sync_copy(r1, e1_o.at[pl.ds(off, n)])
      c2.wait()
      pltpu.sync_copy(r2, e2_o.at[pl.ds(off, n)])
      return carry

    lax.fori_loop(0, n_chunks, body, 0)

  return sc_feed


@functools.lru_cache(maxsize=None)
def _make_sc_side(VA, n, Bb):
  info = plsc.get_sparse_core_info()
  NC, NS = info.num_cores, info.num_subcores
  NW = NC * NS
  b_per_w = Bb // NW
  n_chunks = b_per_w // n
  mesh = plsc.VectorSubcoreMesh(core_axis_name="c", subcore_axis_name="s")

  iota16 = lambda: lax.iota(jnp.int32, 16)

  @functools.partial(
      pl.kernel,
      out_type=(
          jax.ShapeDtypeStruct((Bb, D), jnp.float32),
          jax.ShapeDtypeStruct((Bb, VTP), jnp.float32),
      ),
      mesh=mesh,
      compiler_params=pltpu.CompilerParams(**_SC_PARAMS),
      scratch_types=[
          pltpu.VMEM((n,), jnp.int32),
          pltpu.VMEM((n,), jnp.int32),
          pltpu.VMEM((n, D), jnp.float32),
          pltpu.VMEM((n, D), jnp.int32),
          pltpu.VMEM((n, VTP), jnp.float32),
          pltpu.SemaphoreType.DMA,
          pltpu.SemaphoreType.DMA,
      ],
  )
  def sc_side(feed_id, author_tab, combo, hzero,
              a_o, c_o, idx_v, ai_v, ar, tg, hist, s4, s5):
    wid = lax.axis_index("s") * NC + lax.axis_index("c")
    base = wid * b_per_w

    pltpu.sync_copy(hzero, hist)

    ones16 = jnp.ones((16,), jnp.float32)
    zeros16 = jnp.zeros((16,), jnp.float32)

    def body(j, carry):
      off = base + j * n
      pltpu.sync_copy(feed_id.at[pl.ds(off, n)], idx_v)
      c4 = pltpu.async_copy(combo.at[idx_v], tg, s4)
      c4.wait()
      for g in range(n // 16):
        rows = g * 16 + iota16()
        a16 = plsc.load_gather(tg, [rows, jnp.full((16,), LT, jnp.int32)])
        ai_v[pl.ds(g * 16, 16)] = a16
      c5 = pltpu.async_copy(author_tab.at[ai_v], ar, s5)
      touched = []
      for g in range(n // 16):
        rows = g * 16 + iota16()
        for l in range(LT):
          lsplat = jnp.full((16,), l, jnp.int32)
          t16 = plsc.load_gather(tg, [rows, lsplat])
          plsc.addupdate_scatter(hist, [rows, t16], ones16)
          touched.append((rows, t16))
      c5.wait()
      pltpu.sync_copy(ar, a_o.at[pl.ds(off, n)])
      pltpu.sync_copy(hist, c_o.at[pl.ds(off, n)])
      for rows, t16 in touched:
        plsc.store_scatter(hist, [rows, t16], zeros16)
      return carry

    lax.fori_loop(0, n_chunks, body, 0)

  return sc_side


def _tc_body(e1, e2, ao, cin, w1a, w1b, b1, w2a, w2b, w2c, b2, temb, out_ref):
  fe = (lax.dot_general(e1[...], w1a[...], (((1,), (0,)), ((), ())),
                        preferred_element_type=jnp.float32)
        + lax.dot_general(e2[...], w1b[...], (((1,), (0,)), ((), ())),
                          preferred_element_type=jnp.float32)
        + b1[...])

  S = lax.dot_general(fe, temb[...], (((1,), (1,)), ((), ())),
                      preferred_element_type=jnp.float32)
  S = S * jnp.float32(1.0 / math.sqrt(D))

  c = cin[...][:, :VT]
  bB = c.shape[0]
  iota_v = lax.broadcasted_iota(jnp.int32, (bB, VT), 1)

  S_masked = jnp.where(iota_v > 0, S, jnp.float32(-1e30))
  m = jnp.max(S_masked, axis=1, keepdims=True)
  E = c * jnp.exp(S_masked - m)
  Z = jnp.sum(E, axis=1, keepdims=True)
  good = Z > 0
  E = jnp.where(good, E, c)
  Z = jnp.where(good, Z, jnp.float32(LT))

  att = lax.dot_general(E, temb[...], (((1,), (0,)), ((), ())),
                        preferred_element_type=jnp.float32) / Z

  out = (lax.dot_general(fe, w2a[...], (((1,), (0,)), ((), ())),
                         preferred_element_type=jnp.float32)
         + lax.dot_general(ao[...], w2b[...], (((1,), (0,)), ((), ())),
                           preferred_element_type=jnp.float32)
         + lax.dot_general(att, w2c[...], (((1,), (0,)), ((), ())),
                           preferred_element_type=jnp.float32)
         + b2[...])
  out_ref[...] = out


@functools.lru_cache(maxsize=None)
def _make_tc(bB, Bb):
  grid = (Bb // bB,)
  row_spec = pl.BlockSpec((bB, D), lambda i: (i, 0))
  hist_spec = pl.BlockSpec((bB, VTP), lambda i: (i, 0))
  full = lambda shape: pl.BlockSpec(shape, lambda i: (0,) * len(shape))
  return pl.pallas_call(
      _tc_body,
      grid=grid,
      in_specs=[
          row_spec, row_spec, row_spec, hist_spec,
          full((D, D)), full((D, D)), full((1, D)),
          full((D, D)), full((D, D)), full((D, D)), full((1, D)),
          full((VT, D)),
      ],
      out_specs=row_spec,
      out_shape=jax.ShapeDtypeStruct((Bb, D), jnp.float32),
      compiler_params=pltpu.CompilerParams(
          dimension_semantics=("arbitrary",),
      ),
  )


def kernel(feed_id, feed_emb, w2v_emb, author_emb, tag_emb, side_author,
           side_tags, W1, b1, W2, b2):
  VF = feed_emb.shape[0]
  VA = author_emb.shape[0]
  fid = feed_id.astype(jnp.int32)
  combo = jnp.concatenate(
      [side_tags.astype(jnp.int32), side_author.astype(jnp.int32),
       jnp.zeros((VF, D - LT - 1), jnp.int32)], axis=1)

  n_split = 1
  Bb = B // n_split
  hzero = jnp.zeros((64, VTP), jnp.float32)
  sc_feed = _make_sc_feed(VF, 128, Bb)
  sc_side = _make_sc_side(VA, 64, Bb)
  tc = _make_tc(256, Bb)
  w_args = (W1[:D], W1[D:], b1.reshape((1, D)),
            W2[:D], W2[D:2 * D], W2[2 * D:], b2.reshape((1, D)), tag_emb)

  outs = []
  for i in range(n_split):
    fid_i = fid[i * Bb:(i + 1) * Bb]
    e1, e2 = sc_feed(fid_i, feed_emb, w2v_emb)
    a_out, c = sc_side(fid_i, author_emb, combo, hzero)
    outs.append(tc(e1, e2, a_out, c, *w_args))
  return outs[0] if n_split == 1 else jnp.concatenate(outs, axis=0)


# speedup vs baseline: 1.1481x; 1.1481x over previous
import functools
import math

import jax
import jax.numpy as jnp
from jax import lax
from jax.experimental import pallas as pl
from jax.experimental.pallas import tpu as pltpu
from jax.experimental.pallas import tpu_sc as plsc

B = 16384
D = 128
VT = 1000
VTP = 1024
VW = VTP // 4
LT = 50

_SC_PARAMS = dict(use_tc_tiling_on_sc=True, needs_layout_passes=False)


@functools.lru_cache(maxsize=None)
def _make_sc_feed(VF, n, Bb):
  info = plsc.get_sparse_core_info()
  NC, NS = info.num_cores, info.num_subcores
  NW = NC * NS
  b_per_w = Bb // NW
  n_chunks = b_per_w // n
  mesh = plsc.VectorSubcoreMesh(core_axis_name="c", subcore_axis_name="s")

  @functools.partial(
      pl.kernel,
      out_type=(
          jax.ShapeDtypeStruct((Bb, D), jnp.float32),
          jax.ShapeDtypeStruct((Bb, D), jnp.float32),
      ),
      mesh=mesh,
      compiler_params=pltpu.CompilerParams(**_SC_PARAMS),
      scratch_types=[
          pltpu.VMEM((n,), jnp.int32),
          pltpu.VMEM((n, D), jnp.float32),
          pltpu.VMEM((n, D), jnp.float32),
          pltpu.SemaphoreType.DMA,
          pltpu.SemaphoreType.DMA,
      ],
  )
  def sc_feed(feed_id, feed_tab, w2v_tab, e1_o, e2_o, idx_v, r1, r2, s1, s2):
    wid = lax.axis_index("s") * NC + lax.axis_index("c")
    base = wid * b_per_w

    def body(j, carry):
      off = base + j * n
      pltpu.sync_copy(feed_id.at[pl.ds(off, n)], idx_v)
      c1 = pltpu.async_copy(feed_tab.at[idx_v], r1, s1)
      c2 = pltpu.async_copy(w2v_tab.at[idx_v], r2, s2)
      c1.wait()
      pltpu.sync_copy(r1, e1_o.at[pl.ds(off, n)])
      c2.wait()
      pltpu.sync_copy(r2, e2_o.at[pl.ds(off, n)])
      return carry

    lax.fori_loop(0, n_chunks, body, 0)

  return sc_feed


@functools.lru_cache(maxsize=None)
def _make_sc_side(VA, n, Bb):
  info = plsc.get_sparse_core_info()
  NC, NS = info.num_cores, info.num_subcores
  NW = NC * NS
  b_per_w = Bb // NW
  n_chunks = b_per_w // n
  mesh = plsc.VectorSubcoreMesh(core_axis_name="c", subcore_axis_name="s")

  iota16 = lambda: lax.iota(jnp.int32, 16)

  @functools.partial(
      pl.kernel,
      out_type=(
          jax.ShapeDtypeStruct((Bb, D), jnp.float32),
          jax.ShapeDtypeStruct((Bb, VW), jnp.int32),
      ),
      mesh=mesh,
      compiler_params=pltpu.CompilerParams(**_SC_PARAMS),
      scratch_types=[
          pltpu.VMEM((n,), jnp.int32),
          pltpu.VMEM((n,), jnp.int32),
          pltpu.VMEM((n, D), jnp.float32),
          pltpu.VMEM((n, D), jnp.int32),
          pltpu.VMEM((n, VW), jnp.int32),
          pltpu.SemaphoreType.DMA,
          pltpu.SemaphoreType.DMA,
          pltpu.SemaphoreType.DMA,
      ],
  )
  def sc_side(feed_id, author_tab, sauthor, stags, hzero,
              a_o, c_o, idx_v, ai_v, ar, tg, hist, s3, s4, s5):
    wid = lax.axis_index("s") * NC + lax.axis_index("c")
    base = wid * b_per_w

    pltpu.sync_copy(hzero, hist)

    one = jnp.int32(1)
    zeros16 = jnp.zeros((16,), jnp.int32)

    def body(j, carry):
      off = base + j * n
      pltpu.sync_copy(feed_id.at[pl.ds(off, n)], idx_v)
      c3 = pltpu.async_copy(sauthor.at[idx_v], ai_v, s3)
      c4 = pltpu.async_copy(stags.at[idx_v], tg, s4)
      c3.wait()
      c5 = pltpu.async_copy(author_tab.at[ai_v], ar, s5)
      c4.wait()
      touched = []
      for g in range(n // 16):
        rows = g * 16 + iota16()
        for l in range(LT):
          lsplat = jnp.full((16,), l, jnp.int32)
          t16 = plsc.load_gather(tg, [rows, lsplat])
          w16 = lax.shift_right_logical(t16, 2)
          inc16 = lax.shift_left(one, lax.shift_left(t16 & 3, 3))
          plsc.addupdate_scatter(hist, [rows, w16], inc16)
          touched.append((rows, w16))
      c5.wait()
      pltpu.sync_copy(ar, a_o.at[pl.ds(off, n)])
      pltpu.sync_copy(hist, c_o.at[pl.ds(off, n)])
      for rows, w16 in touched:
        plsc.store_scatter(hist, [rows, w16], zeros16)
      return carry

    lax.fori_loop(0, n_chunks, body, 0)

  return sc_side


def _tc_body(e1, e2, ao, cin, w1a, w1b, b1, w2a, w2b, w2c, b2, tep, out_ref):
  f32 = jnp.float32
  fe = (lax.dot_general(e1[...], w1a[...], (((1,), (0,)), ((), ())),
                        preferred_element_type=f32)
        + lax.dot_general(e2[...], w1b[...], (((1,), (0,)), ((), ())),
                          preferred_element_type=f32)
        + b1[...])

  tep16 = tep[...]
  S = lax.dot_general(fe.astype(jnp.bfloat16), tep16,
                      (((1,), (1,)), ((), ())), preferred_element_type=f32)
  S = S * f32(1.0 / math.sqrt(D))

  cw = cin[...]
  parts = [((lax.shift_right_logical(cw, 8 * j)) & 255).astype(f32)
           for j in range(4)]
  c = jnp.concatenate(parts, axis=1)

  bB = cw.shape[0]
  iota_p = lax.broadcasted_iota(jnp.int32, (bB, VTP), 1)
  S_masked = jnp.where(iota_p > 0, S, f32(-1e30))
  m = jnp.max(S_masked, axis=1, keepdims=True)
  E = c * jnp.exp(S_masked - m)
  Z = jnp.sum(E, axis=1, keepdims=True)
  good = Z > 0
  E = jnp.where(good, E, c)
  Z = jnp.where(good, Z, f32(LT))

  att = lax.dot_general(E.astype(jnp.bfloat16), tep16,
                        (((1,), (0,)), ((), ())),
                        preferred_element_type=f32) / Z

  out = (lax.dot_general(fe, w2a[...], (((1,), (0,)), ((), ())),
                         preferred_element_type=f32)
         + lax.dot_general(ao[...], w2b[...], (((1,), (0,)), ((), ())),
                           preferred_element_type=f32)
         + lax.dot_general(att, w2c[...], (((1,), (0,)), ((), ())),
                           preferred_element_type=f32)
         + b2[...])
  out_ref[...] = out


@functools.lru_cache(maxsize=None)
def _make_tc(bB, Bb):
  grid = (Bb // bB,)
  row_spec = pl.BlockSpec((bB, D), lambda i: (i, 0))
  hist_spec = pl.BlockSpec((bB, VW), lambda i: (i, 0))
  full = lambda shape: pl.BlockSpec(shape, lambda i: (0,) * len(shape))
  return pl.pallas_call(
      _tc_body,
      grid=grid,
      in_specs=[
          row_spec, row_spec, row_spec, hist_spec,
          full((D, D)), full((D, D)), full((1, D)),
          full((D, D)), full((D, D)), full((D, D)), full((1, D)),
          full((VTP, D)),
      ],
      out_specs=row_spec,
      out_shape=jax.ShapeDtypeStruct((Bb, D), jnp.float32),
      compiler_params=pltpu.CompilerParams(
          dimension_semantics=("arbitrary",),
      ),
  )


def kernel(feed_id, feed_emb, w2v_emb, author_emb, tag_emb, side_author,
           side_tags, W1, b1, W2, b2):
  VF = feed_emb.shape[0]
  VA = author_emb.shape[0]
  fid = feed_id.astype(jnp.int32)
  sa_flat = side_author.reshape((VF,)).astype(jnp.int32)
  st_pad = jnp.pad(side_tags.astype(jnp.int32), ((0, 0), (0, D - LT)))

  p = jnp.arange(VTP)
  perm = 4 * (p % VW) + p // VW
  tep = jnp.concatenate(
      [tag_emb, jnp.zeros((VTP - VT, D), jnp.float32)], axis=0)[perm]
  tep16 = tep.astype(jnp.bfloat16)

  n_split = 1
  Bb = B // n_split
  hzero = jnp.zeros((128, VW), jnp.int32)
  sc_feed = _make_sc_feed(VF, 128, Bb)
  sc_side = _make_sc_side(VA, 128, Bb)
  tc = _make_tc(256, Bb)
  w_args = (W1[:D], W1[D:], b1.reshape((1, D)),
            W2[:D], W2[D:2 * D], W2[2 * D:], b2.reshape((1, D)), tep16)

  outs = []
  for i in range(n_split):
    fid_i = fid[i * Bb:(i + 1) * Bb]
    e1, e2 = sc_feed(fid_i, feed_emb, w2v_emb)
    a_out, c = sc_side(fid_i, author_emb, sa_flat, st_pad, hzero)
    outs.append(tc(e1, e2, a_out, c, *w_args))
  return outs[0] if n_split == 1 else jnp.concatenate(outs, axis=0)
